# baseline (speedup 1.0000x reference)
import functools
import jax
import jax.numpy as jnp
from jax import lax
from jax.experimental import pallas as pl
from jax.experimental.pallas import tpu as pltpu
from jax.experimental.pallas import tpu_sc as plsc

D = 128
_NC = 2
_NS = 16
_NW = _NC * _NS
_CHUNK = 128


def _matmul(x, W):
  n = x.shape[0]
  blk = 1000

  def body(x_ref, w_ref, o_ref):
    o_ref[...] = jnp.dot(x_ref[...], w_ref[...],
                         preferred_element_type=jnp.float32)

  return pl.pallas_call(
      body,
      grid=(n // blk,),
      in_specs=[pl.BlockSpec((blk, D), lambda i: (i, 0)),
                pl.BlockSpec((D, D), lambda i: (0, 0))],
      out_specs=pl.BlockSpec((blk, D), lambda i: (i, 0)),
      out_shape=jax.ShapeDtypeStruct((n, D), jnp.float32),
  )(x, W)


def _aggregate(support, row3, col3, n_pad, ch):
  rows_per_tile = n_pad // _NS
  mesh = plsc.VectorSubcoreMesh(core_axis_name="c", subcore_axis_name="s")
  zeros = jnp.zeros((rows_per_tile, D), jnp.float32)

  hch = ch // 2

  @functools.partial(
      pl.kernel,
      mesh=mesh,
      out_type=jax.ShapeDtypeStruct((_NC, n_pad, D), jnp.float32),
      scratch_types=[
          pltpu.VMEM((hch, _CHUNK), jnp.int32),
          pltpu.VMEM((hch, _CHUNK), jnp.int32),
          pltpu.VMEM((2, _CHUNK, D), jnp.float32),
          pltpu.VMEM_SHARED((n_pad, D), jnp.float32),
          pltpu.SemaphoreType.DMA,
          pltpu.SemaphoreType.DMA,
      ],
  )
  def agg(support_hbm, row_hbm, col_hbm, zero_hbm, out_hbm,
          row_v, col_v, buf, acc, gsem0, gsem1):
    c = lax.axis_index("c")
    s = lax.axis_index("s")
    wid = s * _NC + c
    base = s * rows_per_tile

    pltpu.sync_copy(zero_hbm, acc.at[pl.ds(base, rows_per_tile)])
    pltpu.sync_copy(acc.at[pl.ds(base + rows_per_tile - 8, 8)],
                    buf.at[0].at[pl.ds(0, 8)])
    plsc.subcore_barrier()

    for h in range(2):
      off = h * hch
      pltpu.sync_copy(row_hbm.at[wid].at[pl.ds(off, hch)], row_v)
      pltpu.sync_copy(col_hbm.at[wid].at[pl.ds(off, hch)], col_v)

      def wait_g(j, bi, sem):
        pltpu.make_async_copy(support_hbm.at[col_v.at[j]],
                              buf.at[bi], sem).wait()

      pltpu.async_copy(support_hbm.at[col_v.at[0]], buf.at[0], gsem0)
      pltpu.async_copy(support_hbm.at[col_v.at[1]], buf.at[1], gsem1)

      def body(jj, carry):
        j0 = jj * 2
        wait_g(j0, 0, gsem0)
        pltpu.sync_copy(buf.at[0], acc.at[row_v.at[j0]], add=True)
        pltpu.async_copy(support_hbm.at[col_v.at[j0 + 2]], buf.at[0],
                         gsem0)
        wait_g(j0 + 1, 1, gsem1)
        pltpu.sync_copy(buf.at[1], acc.at[row_v.at[j0 + 1]], add=True)
        pltpu.async_copy(support_hbm.at[col_v.at[j0 + 3]], buf.at[1],
                         gsem1)
        return carry

      lax.fori_loop(0, hch // 2 - 1, body, 0)

      wait_g(hch - 2, 0, gsem0)
      pltpu.sync_copy(buf.at[0], acc.at[row_v.at[hch - 2]], add=True)
      wait_g(hch - 1, 1, gsem1)
      pltpu.sync_copy(buf.at[1], acc.at[row_v.at[hch - 1]], add=True)

    pltpu.sync_copy(acc.at[pl.ds(base, 8)], buf.at[1].at[pl.ds(0, 8)])
    plsc.subcore_barrier()
    pltpu.sync_copy(acc.at[pl.ds(base, rows_per_tile)],
                    out_hbm.at[c].at[pl.ds(base, rows_per_tile)])

  return agg(support, row3, col3, zeros)


def _combine(parts, b2, n):
  blk = 1000

  def body(p_ref, b_ref, o_ref):
    o_ref[...] = p_ref[0] + p_ref[1] + b_ref[...]

  return pl.pallas_call(
      body,
      grid=(n // blk,),
      in_specs=[pl.BlockSpec((_NC, blk, D), lambda i: (0, i, 0)),
                pl.BlockSpec((1, D), lambda i: (0, 0))],
      out_specs=pl.BlockSpec((blk, D), lambda i: (i, 0)),
      out_shape=jax.ShapeDtypeStruct((n, D), jnp.float32),
  )(parts, b2)


def kernel(x, edge_index, W, b):
  n = x.shape[0]
  e = edge_index.shape[1]
  ch = -(-e // (_NW * _CHUNK))
  ch = -(-ch // 4) * 4
  e_pad = _NW * ch * _CHUNK
  n_pad = -(-(n + 1) // (_NS * 8)) * (_NS * 8)

  row = edge_index[0].astype(jnp.int32)
  col = edge_index[1].astype(jnp.int32)
  pad = e_pad - e
  if pad:
    pad_rows = n + jnp.arange(pad, dtype=jnp.int32) % (n_pad - n)
    row = jnp.concatenate([row, pad_rows])
    col = jnp.concatenate([col, jnp.zeros((pad,), jnp.int32)])
  row3 = row.reshape(_NW, ch, _CHUNK)
  col3 = col.reshape(_NW, ch, _CHUNK)

  support = _matmul(x, W)
  parts = _aggregate(support, row3, col3, n_pad, ch)
  return _combine(parts, b.reshape(1, D), n)


# speedup vs baseline: 3.0599x; 3.0599x over previous
import functools
import jax
import jax.numpy as jnp
from jax import lax
from jax.experimental import pallas as pl
from jax.experimental.pallas import tpu as pltpu
from jax.experimental.pallas import tpu_sc as plsc

D = 128
_NC = 2
_NS = 16
_NW = _NC * _NS
_CHUNK = 128


def _matmul(x, W):
  n = x.shape[0]
  blk = 1000

  def body(x_ref, w_ref, o_ref):
    o_ref[...] = jnp.dot(x_ref[...], w_ref[...],
                         preferred_element_type=jnp.float32)

  return pl.pallas_call(
      body,
      grid=(n // blk,),
      in_specs=[pl.BlockSpec((blk, D), lambda i: (i, 0)),
                pl.BlockSpec((D, D), lambda i: (0, 0))],
      out_specs=pl.BlockSpec((blk, D), lambda i: (i, 0)),
      out_shape=jax.ShapeDtypeStruct((n, D), jnp.float32),
  )(x, W)


def _aggregate(support, row3, col3, n_pad, ch):
  rows_per_tile = n_pad // _NS
  mesh = plsc.VectorSubcoreMesh(core_axis_name="c", subcore_axis_name="s")
  zeros = jnp.zeros((rows_per_tile, D), jnp.float32)

  hch = ch // 2

  @functools.partial(
      pl.kernel,
      mesh=mesh,
      out_type=jax.ShapeDtypeStruct((_NC, n_pad, D), jnp.float32),
      scratch_types=[
          pltpu.VMEM((hch, _CHUNK), jnp.int32),
          pltpu.VMEM((hch, _CHUNK), jnp.int32),
          pltpu.VMEM((2, _CHUNK, D), jnp.float32),
          pltpu.VMEM_SHARED((n_pad, D), jnp.float32),
          pltpu.SemaphoreType.DMA,
          pltpu.SemaphoreType.DMA,
      ],
  )
  def agg(support_hbm, row_hbm, col_hbm, zero_hbm, out_hbm,
          row_v, col_v, buf, acc, gsem0, gsem1):
    c = lax.axis_index("c")
    s = lax.axis_index("s")
    wid = s * _NC + c
    base = s * rows_per_tile

    pltpu.sync_copy(zero_hbm, acc.at[pl.ds(base, rows_per_tile)])
    pltpu.sync_copy(acc.at[pl.ds(base + rows_per_tile - 8, 8)],
                    buf.at[0].at[pl.ds(0, 8)])
    plsc.subcore_barrier()

    for h in range(2):
      off = h * hch
      pltpu.sync_copy(row_hbm.at[wid].at[pl.ds(off, hch)], row_v)
      pltpu.sync_copy(col_hbm.at[wid].at[pl.ds(off, hch)], col_v)

      def wait_g(j, bi, sem):
        pltpu.make_async_copy(support_hbm.at[col_v.at[j]],
                              buf.at[bi], sem).wait()

      pltpu.async_copy(support_hbm.at[col_v.at[0]], buf.at[0], gsem0)
      pltpu.async_copy(support_hbm.at[col_v.at[1]], buf.at[1], gsem1)

      def body(jj, carry):
        j0 = jj * 2
        wait_g(j0, 0, gsem0)
        pltpu.sync_copy(buf.at[0], acc.at[row_v.at[j0]], add=True)
        pltpu.async_copy(support_hbm.at[col_v.at[j0 + 2]], buf.at[0],
                         gsem0)
        wait_g(j0 + 1, 1, gsem1)
        pltpu.sync_copy(buf.at[1], acc.at[row_v.at[j0 + 1]], add=True)
        pltpu.async_copy(support_hbm.at[col_v.at[j0 + 3]], buf.at[1],
                         gsem1)
        return carry

      lax.fori_loop(0, hch // 2 - 1, body, 0)

      wait_g(hch - 2, 0, gsem0)
      pltpu.sync_copy(buf.at[0], acc.at[row_v.at[hch - 2]], add=True)
      wait_g(hch - 1, 1, gsem1)
      pltpu.sync_copy(buf.at[1], acc.at[row_v.at[hch - 1]], add=True)

    pltpu.sync_copy(acc.at[pl.ds(base, 8)], buf.at[1].at[pl.ds(0, 8)])
    plsc.subcore_barrier()
    pltpu.sync_copy(acc.at[pl.ds(base, rows_per_tile)],
                    out_hbm.at[c].at[pl.ds(base, rows_per_tile)])

  return agg(support, row3, col3, zeros)


def _combine(parts, b2, n):
  blk = 1000

  def body(p_ref, b_ref, o_ref):
    o_ref[...] = p_ref[0] + p_ref[1] + b_ref[...]

  return pl.pallas_call(
      body,
      grid=(n // blk,),
      in_specs=[pl.BlockSpec((_NC, blk, D), lambda i: (0, i, 0)),
                pl.BlockSpec((1, D), lambda i: (0, 0))],
      out_specs=pl.BlockSpec((blk, D), lambda i: (i, 0)),
      out_shape=jax.ShapeDtypeStruct((n, D), jnp.float32),
  )(parts, b2)


def kernel(x, edge_index, W, b):
  n = x.shape[0]
  e = edge_index.shape[1]
  ch = -(-e // (_NW * _CHUNK))
  ch = -(-ch // 4) * 4
  e_pad = _NW * ch * _CHUNK
  n_pad = -(-(n + 1) // (_NS * 8)) * (_NS * 8)

  row = edge_index[0].astype(jnp.int32)
  col = edge_index[1].astype(jnp.int32)
  pad = e_pad - e
  if pad:
    pad_idx = jnp.arange(pad, dtype=jnp.int32)
    row = jnp.concatenate([row, n + pad_idx % (n_pad - n)])
    col = jnp.concatenate([col, pad_idx * 79 % n])
  row3 = row.reshape(_NW, ch, _CHUNK)
  col3 = col.reshape(_NW, ch, _CHUNK)

  support = _matmul(x, W)
  parts = _aggregate(support, row3, col3, n_pad, ch)
  return _combine(parts, b.reshape(1, D), n)
